# speedup vs baseline: 3.2749x; 3.2749x over previous
import functools

import jax
import jax.numpy as jnp
from jax import lax
from jax.experimental import pallas as pl
from jax.experimental.pallas import tpu as pltpu
from jax.experimental.pallas import tpu_sc as plsc

EMBED_DIM = 128
ATTN_DIM = 768

_NC = 2
_NS = 16
_NW = _NC * _NS

_CHUNK = 128


def _make_sc_gather(n_tokens: int):
    per_w = n_tokens // _NW
    chunks = per_w // _CHUNK

    mesh = plsc.VectorSubcoreMesh(core_axis_name="c", subcore_axis_name="s")

    @functools.partial(
        pl.kernel,
        mesh=mesh,
        out_type=jax.ShapeDtypeStruct((n_tokens, EMBED_DIM), jnp.float32),
        scratch_types=[
            pltpu.VMEM((chunks, _CHUNK), jnp.int32),
            pltpu.VMEM((_CHUNK, EMBED_DIM), jnp.float32),
            pltpu.SemaphoreType.DMA,
        ],
    )
    def sc_gather(table_hbm, idx_hbm, out_hbm, idx_v, rows_v, gsem):
        wid = lax.axis_index("s") * _NC + lax.axis_index("c")
        row_base = wid * chunks
        pltpu.sync_copy(idx_hbm.at[pl.ds(row_base, chunks)], idx_v)

        def body(g, carry):
            pltpu.async_copy(table_hbm.at[idx_v.at[g]], rows_v, gsem).wait()
            tok = (row_base + g) * _CHUNK
            pltpu.sync_copy(rows_v, out_hbm.at[pl.ds(tok, _CHUNK)])
            return carry

        lax.fori_loop(0, chunks, body, 0)

    return sc_gather


def _mm_body(emb_ref, w_ref, out_ref):
    out_ref[...] = lax.dot_general(
        emb_ref[...], w_ref[...],
        dimension_numbers=(((1,), (1,)), ((), ())),
        preferred_element_type=jnp.float32,
    )


def _project(emb, W, tile: int):
    n = emb.shape[0]
    return pl.pallas_call(
        _mm_body,
        grid=(n // tile,),
        in_specs=[
            pl.BlockSpec((tile, EMBED_DIM), lambda i: (i, 0)),
            pl.BlockSpec((ATTN_DIM, EMBED_DIM), lambda i: (0, 0)),
        ],
        out_specs=pl.BlockSpec((tile, ATTN_DIM), lambda i: (i, 0)),
        out_shape=jax.ShapeDtypeStruct((n, ATTN_DIM), jnp.float32),
        compiler_params=pltpu.CompilerParams(
            dimension_semantics=("parallel",),
        ),
    )(emb, W)


def kernel(x, table, W):
    b, s = x.shape
    n = b * s
    idx2d = x.reshape(n // _CHUNK, _CHUNK).astype(jnp.int32)
    emb = _make_sc_gather(n)(table, idx2d)
    out = _project(emb, W, tile=1024)
    return out.reshape(b, s, ATTN_DIM)
